# baseline (speedup 1.0000x reference)
import jax
import jax.numpy as jnp
from jax.experimental import pallas as pl
from jax.experimental.pallas import tpu as pltpu

_N_E = 8192
_HALF = _N_E // 2
_D = 32
_T = 128
_N_TOK = 8192
_GRID = _N_TOK // _T
_BETA = 0.25


def _vq_body(z_ref, e_ref, enc_ref, cb_ref, zq_ref, loss_ref, perp_ref,
             acc_ref, cnt_ref, b2_ref):
    i = pl.program_id(0)
    z = z_ref[...]
    e = e_ref[...]

    @pl.when(i == 0)
    def _precompute():
        b2_ref[...] = jnp.sum(e * e, axis=1)[None, :]

    a2 = jnp.sum(z * z, axis=1, keepdims=True)
    b2 = b2_ref[...]
    zb = z.astype(jnp.bfloat16)
    eb = e.astype(jnp.bfloat16)
    m = jax.lax.dot_general(zb, eb, (((1,), (1,)), ((), ())),
                            preferred_element_type=jnp.float32)
    d = (a2 + b2) - 2.0 * m

    d_a = d[:, :_HALF]
    d_b = d[:, _HALF:]
    v_a = jnp.min(d_a, axis=1, keepdims=True)
    v_b = jnp.min(d_b, axis=1, keepdims=True)
    v_a_bf = v_a.astype(jnp.bfloat16).astype(jnp.float32)
    choose_b = v_b < v_a_bf
    d_sel = jnp.where(choose_b, d_b, d_a)
    v_sel = jnp.where(choose_b, v_b, v_a)
    lanes_h = jax.lax.broadcasted_iota(jnp.int32, (_T, _HALF), 1)
    i_loc = jnp.min(jnp.where(d_sel == v_sel, lanes_h, _N_E), axis=1)
    idx = i_loc + jnp.where(choose_b[:, 0], _HALF, 0)
    cb_ref[...] = idx

    lanes = jax.lax.broadcasted_iota(jnp.int32, (_T, _N_E), 1)

    enc = jnp.where(lanes == idx[:, None], 1.0, 0.0).astype(jnp.float32)
    enc_ref[...] = enc

    zq = jax.lax.dot_general(enc.astype(jnp.bfloat16), eb,
                             (((1,), (0,)), ((), ())),
                             preferred_element_type=jnp.float32)
    zq_ref[...] = z + (zq - z)

    @pl.when(i == 0)
    def _init():
        acc_ref[0] = 0.0
        cnt_ref[...] = jnp.zeros_like(cnt_ref)

    diff = zq - z
    acc_ref[0] += jnp.sum(diff * diff)
    cnt_ref[...] += jnp.sum(enc, axis=0, keepdims=True)

    @pl.when(i == _GRID - 1)
    def _fin():
        mse = acc_ref[0] / jnp.float32(_N_TOK * _D)
        loss_ref[...] = jnp.reshape(mse + _BETA * mse, (1, 1))
        e_mean = cnt_ref[...] * jnp.float32(1.0 / _N_TOK)
        ent = jnp.sum(e_mean * jnp.log(e_mean + 1e-10))
        perp_ref[...] = jnp.reshape(jnp.exp(-ent), (1, 1))


def kernel(z, embedding_weight):
    B, C, H, W = z.shape
    zp = jnp.transpose(z, (0, 2, 3, 1))
    z_flat = zp.reshape(-1, _D)

    enc, cb, zq_st, loss, perp = pl.pallas_call(
        _vq_body,
        grid=(_GRID,),
        in_specs=[
            pl.BlockSpec((_T, _D), lambda i: (i, 0)),
            pl.BlockSpec((_N_E, _D), lambda i: (0, 0)),
        ],
        out_specs=[
            pl.BlockSpec((_T, _N_E), lambda i: (i, 0)),
            pl.BlockSpec((_T,), lambda i: (i,)),
            pl.BlockSpec((_T, _D), lambda i: (i, 0)),
            pl.BlockSpec((1, 1), lambda i: (0, 0)),
            pl.BlockSpec((1, 1), lambda i: (0, 0)),
        ],
        out_shape=[
            jax.ShapeDtypeStruct((_N_TOK, _N_E), jnp.float32),
            jax.ShapeDtypeStruct((_N_TOK,), jnp.int32),
            jax.ShapeDtypeStruct((_N_TOK, _D), jnp.float32),
            jax.ShapeDtypeStruct((1, 1), jnp.float32),
            jax.ShapeDtypeStruct((1, 1), jnp.float32),
        ],
        scratch_shapes=[
            pltpu.SMEM((1,), jnp.float32),
            pltpu.VMEM((1, _N_E), jnp.float32),
            pltpu.VMEM((1, _N_E), jnp.float32),
        ],
        compiler_params=pltpu.CompilerParams(
            dimension_semantics=("arbitrary",)),
    )(z_flat, embedding_weight)

    z_q_out = jnp.transpose(zq_st.reshape(B, H, W, C), (0, 3, 1, 2))
    return (loss.reshape(()), z_q_out, perp.reshape(()), enc, cb)


# speedup vs baseline: 1.0706x; 1.0706x over previous
import jax
import jax.numpy as jnp
from jax.experimental import pallas as pl
from jax.experimental.pallas import tpu as pltpu

_N_E = 8192
_HALF = _N_E // 2
_D = 32
_T = 128
_N_TOK = 8192
_GRID = _N_TOK // _T
_BETA = 0.25


def _vq_body(z_ref, e_ref, enc_ref, cb_ref, zq_ref, loss_ref, perp_ref,
             acc_ref, cnt_ref, b2_ref):
    i = pl.program_id(0)
    z = z_ref[...]
    e = e_ref[...]

    @pl.when(i == 0)
    def _precompute():
        b2_ref[...] = jnp.sum(e * e, axis=1)[None, :]

    a2 = jnp.sum(z * z, axis=1, keepdims=True)
    b2 = b2_ref[...]
    zb = z.astype(jnp.bfloat16)
    eb = e.astype(jnp.bfloat16)
    m = jax.lax.dot_general(zb, eb, (((1,), (1,)), ((), ())),
                            preferred_element_type=jnp.float32)
    d = (a2 + b2) - 2.0 * m

    d_a = d[:, :_HALF]
    d_b = d[:, _HALF:]
    v_a = jnp.min(d_a, axis=1, keepdims=True)
    v_b = jnp.min(d_b, axis=1, keepdims=True)
    v_a_bf = v_a.astype(jnp.bfloat16).astype(jnp.float32)
    choose_b = v_b < v_a_bf
    d_sel = jnp.where(choose_b, d_b, d_a)
    v_sel = jnp.where(choose_b, v_b, v_a)
    lanes_h = jax.lax.broadcasted_iota(jnp.int32, (_T, _HALF), 1)
    i_loc = jnp.min(jnp.where(d_sel == v_sel, lanes_h, _N_E), axis=1)
    idx = i_loc + jnp.where(choose_b[:, 0], _HALF, 0)
    cb_ref[...] = idx

    lanes = jax.lax.broadcasted_iota(jnp.int32, (_T, _N_E), 1)

    enc = jnp.where(lanes == idx[:, None], 1.0, 0.0).astype(jnp.float32)
    enc_ref[...] = enc

    zq = jax.lax.dot_general(enc, e, (((1,), (0,)), ((), ())),
                             preferred_element_type=jnp.float32)
    zq_ref[...] = z + (zq - z)

    @pl.when(i == 0)
    def _init():
        acc_ref[0] = 0.0
        cnt_ref[...] = jnp.zeros_like(cnt_ref)

    diff = zq - z
    acc_ref[0] += jnp.sum(diff * diff)
    cnt_ref[...] += jnp.sum(enc, axis=0, keepdims=True)

    @pl.when(i == _GRID - 1)
    def _fin():
        mse = acc_ref[0] / jnp.float32(_N_TOK * _D)
        loss_ref[...] = jnp.reshape(mse + _BETA * mse, (1, 1))
        e_mean = cnt_ref[...] * jnp.float32(1.0 / _N_TOK)
        ent = jnp.sum(e_mean * jnp.log(e_mean + 1e-10))
        perp_ref[...] = jnp.reshape(jnp.exp(-ent), (1, 1))


def kernel(z, embedding_weight):
    B, C, H, W = z.shape
    zp = jnp.transpose(z, (0, 2, 3, 1))
    z_flat = zp.reshape(-1, _D)

    enc, cb, zq_st, loss, perp = pl.pallas_call(
        _vq_body,
        grid=(_GRID,),
        in_specs=[
            pl.BlockSpec((_T, _D), lambda i: (i, 0)),
            pl.BlockSpec((_N_E, _D), lambda i: (0, 0)),
        ],
        out_specs=[
            pl.BlockSpec((_T, _N_E), lambda i: (i, 0)),
            pl.BlockSpec((_T,), lambda i: (i,)),
            pl.BlockSpec((_T, _D), lambda i: (i, 0)),
            pl.BlockSpec((1, 1), lambda i: (0, 0)),
            pl.BlockSpec((1, 1), lambda i: (0, 0)),
        ],
        out_shape=[
            jax.ShapeDtypeStruct((_N_TOK, _N_E), jnp.float32),
            jax.ShapeDtypeStruct((_N_TOK,), jnp.int32),
            jax.ShapeDtypeStruct((_N_TOK, _D), jnp.float32),
            jax.ShapeDtypeStruct((1, 1), jnp.float32),
            jax.ShapeDtypeStruct((1, 1), jnp.float32),
        ],
        scratch_shapes=[
            pltpu.SMEM((1,), jnp.float32),
            pltpu.VMEM((1, _N_E), jnp.float32),
            pltpu.VMEM((1, _N_E), jnp.float32),
        ],
        compiler_params=pltpu.CompilerParams(
            dimension_semantics=("arbitrary",)),
    )(z_flat, embedding_weight)

    z_q_out = jnp.transpose(zq_st.reshape(B, H, W, C), (0, 3, 1, 2))
    return (loss.reshape(()), z_q_out, perp.reshape(()), enc, cb)
